# Initial kernel scaffold; baseline (speedup 1.0000x reference)
import jax
import jax.numpy as jnp
from jax.experimental import pallas as pl


def kernel(x, embeddings):
    raise NotImplementedError("write your pallas kernel here")


# speedup vs baseline: 2.1435x; 2.1435x over previous
import functools

import jax
import jax.numpy as jnp
from jax import lax
from jax.experimental import pallas as pl
from jax.experimental.pallas import tpu as pltpu
from jax.experimental.pallas import tpu_sc as plsc

V_MIN, V_MAX = -6.0, 6.0
BATCH, INPUT_DIM = 4096, 100
NUM_EMB, EMB_DIM = 1000, 64

NUM_CORES, NUM_SUBCORES, LANES = 2, 16, 16
NW = NUM_CORES * NUM_SUBCORES
N_ELEM = BATCH * INPUT_DIM
PER_W = N_ELEM // NW
CHUNK = 256
N_CHUNKS = PER_W // CHUNK
GROUPS = CHUNK // LANES


def _make_body():
    mesh = plsc.VectorSubcoreMesh(core_axis_name="c", subcore_axis_name="s")

    @functools.partial(
        pl.kernel,
        mesh=mesh,
        out_type=jax.ShapeDtypeStruct((N_ELEM * EMB_DIM,), jnp.float32),
        scratch_types=[
            pltpu.VMEM((NUM_EMB * EMB_DIM,), jnp.float32),
            pltpu.VMEM((CHUNK,), jnp.float32),
            pltpu.VMEM((CHUNK * EMB_DIM,), jnp.float32),
        ],
        compiler_params=pltpu.CompilerParams(needs_layout_passes=False),
    )
    def body(x_hbm, emb_hbm, out_hbm, table_v, x_v, out_v):
        wid = lax.axis_index("s") * NUM_CORES + lax.axis_index("c")
        pltpu.sync_copy(emb_hbm, table_v)
        lane64 = lax.iota(jnp.int32, LANES) * EMB_DIM
        scale = jnp.float32(NUM_EMB - 1)

        def chunk_body(ci, carry):
            base = wid * PER_W + ci * CHUNK
            pltpu.sync_copy(x_hbm.at[pl.ds(base, CHUNK)], x_v)

            def group_body(g, carry2):
                xv = x_v[pl.ds(g * LANES, LANES)]
                xs = (xv - V_MIN) / (V_MAX - V_MIN) * scale
                xs = jnp.minimum(jnp.maximum(xs, 0.0), scale)
                idx = xs.astype(jnp.int32)
                w_hi = xs - idx.astype(jnp.float32)
                w_lo = 1.0 - w_hi
                b_lo = idx * EMB_DIM
                b_hi = jnp.minimum(idx + 1, NUM_EMB - 1) * EMB_DIM
                o_base = g * (LANES * EMB_DIM) + lane64
                for c in range(EMB_DIM):
                    a = plsc.load_gather(table_v, [b_lo + c])
                    b = plsc.load_gather(table_v, [b_hi + c])
                    o = w_lo * a + w_hi * b
                    plsc.store_scatter(out_v, [o_base + c], o)
                return carry2

            lax.fori_loop(0, GROUPS, group_body, 0)
            pltpu.sync_copy(out_v,
                            out_hbm.at[pl.ds(base * EMB_DIM, CHUNK * EMB_DIM)])
            return carry

        lax.fori_loop(0, N_CHUNKS, chunk_body, 0)

    return body


_body = _make_body()


@jax.jit
def kernel(x, embeddings):
    out = _body(x.reshape(-1), embeddings.reshape(-1))
    return out.reshape(BATCH, INPUT_DIM * EMB_DIM)


# speedup vs baseline: 3.7756x; 1.7614x over previous
import functools

import jax
import jax.numpy as jnp
from jax import lax
from jax.experimental import pallas as pl
from jax.experimental.pallas import tpu as pltpu
from jax.experimental.pallas import tpu_sc as plsc

V_MIN, V_MAX = -6.0, 6.0
BATCH, INPUT_DIM = 4096, 100
NUM_EMB, EMB_DIM = 1000, 64

NUM_CORES, NUM_SUBCORES, LANES = 2, 16, 16
NW = NUM_CORES * NUM_SUBCORES
N_ELEM = BATCH * INPUT_DIM
PER_W = N_ELEM // NW
CHUNK = 256
N_CHUNKS = PER_W // CHUNK
GROUPS = CHUNK // LANES


def _make_body():
    mesh = plsc.VectorSubcoreMesh(core_axis_name="c", subcore_axis_name="s")

    @functools.partial(
        pl.kernel,
        mesh=mesh,
        out_type=jax.ShapeDtypeStruct((N_ELEM * EMB_DIM,), jnp.float32),
        scratch_types=[
            pltpu.VMEM((NUM_EMB * EMB_DIM,), jnp.float32),
            pltpu.VMEM((CHUNK,), jnp.float32),
            pltpu.VMEM((CHUNK * EMB_DIM,), jnp.float32),
        ],
        compiler_params=pltpu.CompilerParams(needs_layout_passes=False),
    )
    def body(x_hbm, emb_hbm, out_hbm, table_v, x_v, out_v):
        wid = lax.axis_index("s") * NUM_CORES + lax.axis_index("c")
        pltpu.sync_copy(emb_hbm, table_v)
        lane64 = lax.iota(jnp.int32, LANES) * EMB_DIM
        scale = jnp.float32(NUM_EMB - 1)

        def chunk_body(ci, carry):
            base = wid * PER_W + ci * CHUNK
            pltpu.sync_copy(x_hbm.at[pl.ds(base, CHUNK)], x_v)

            def group_body(g, carry2):
                xv = x_v[pl.ds(g * LANES, LANES)]
                xs = (xv - V_MIN) / (V_MAX - V_MIN) * scale
                xs = jnp.minimum(jnp.maximum(xs, 0.0), scale)
                idx = xs.astype(jnp.int32)
                w_hi = xs - idx.astype(jnp.float32)
                w_lo = 1.0 - w_hi
                b_lo = idx * EMB_DIM
                b_hi = jnp.minimum(idx + 1, NUM_EMB - 1) * EMB_DIM
                o_base = g * (LANES * EMB_DIM) + lane64

                @plsc.parallel_loop(0, EMB_DIM, unroll=8)
                def cbody(c):
                    a = plsc.load_gather(table_v, [b_lo + c])
                    b = plsc.load_gather(table_v, [b_hi + c])
                    o = w_lo * a + w_hi * b
                    plsc.store_scatter(out_v, [o_base + c], o)

                return carry2

            lax.fori_loop(0, GROUPS, group_body, 0)
            pltpu.sync_copy(out_v,
                            out_hbm.at[pl.ds(base * EMB_DIM, CHUNK * EMB_DIM)])
            return carry

        lax.fori_loop(0, N_CHUNKS, chunk_body, 0)

    return body


_body = _make_body()


@jax.jit
def kernel(x, embeddings):
    out = _body(x.reshape(-1), embeddings.reshape(-1))
    return out.reshape(BATCH, INPUT_DIM * EMB_DIM)


# speedup vs baseline: 4.0099x; 1.0620x over previous
import functools

import jax
import jax.numpy as jnp
from jax import lax
from jax.experimental import pallas as pl
from jax.experimental.pallas import tpu as pltpu
from jax.experimental.pallas import tpu_sc as plsc

V_MIN, V_MAX = -6.0, 6.0
BATCH, INPUT_DIM = 4096, 100
NUM_EMB, EMB_DIM = 1000, 64

NUM_CORES, NUM_SUBCORES, LANES = 2, 16, 16
NW = NUM_CORES * NUM_SUBCORES
N_ELEM = BATCH * INPUT_DIM
PER_W = N_ELEM // NW
CHUNK = 256
N_CHUNKS = PER_W // CHUNK
N_SUPER = N_CHUNKS // 2
GROUPS = CHUNK // LANES
OUT_W = CHUNK * EMB_DIM


def _make_body():
    mesh = plsc.VectorSubcoreMesh(core_axis_name="c", subcore_axis_name="s")

    @functools.partial(
        pl.kernel,
        mesh=mesh,
        out_type=jax.ShapeDtypeStruct((N_ELEM * EMB_DIM,), jnp.float32),
        scratch_types=[
            pltpu.VMEM((NUM_EMB * EMB_DIM,), jnp.float32),
            pltpu.VMEM((PER_W,), jnp.float32),
            pltpu.VMEM((OUT_W,), jnp.float32),
            pltpu.VMEM((OUT_W,), jnp.float32),
            pltpu.SemaphoreType.DMA,
            pltpu.SemaphoreType.DMA,
        ],
        compiler_params=pltpu.CompilerParams(needs_layout_passes=False),
    )
    def body(x_hbm, emb_hbm, out_hbm, table_v, x_v, out0_v, out1_v, sem0, sem1):
        wid = lax.axis_index("s") * NUM_CORES + lax.axis_index("c")
        span = wid * PER_W
        pltpu.sync_copy(emb_hbm, table_v)
        pltpu.sync_copy(x_hbm.at[pl.ds(span, PER_W)], x_v)
        lane64 = lax.iota(jnp.int32, LANES) * EMB_DIM
        scale = jnp.float32(NUM_EMB - 1)
        sems = (sem0, sem1)
        bufs = (out0_v, out1_v)

        def run_chunk(ci, buf_v, sem):
            off = ci * CHUNK

            def group_body(g, carry2):
                xv = x_v[pl.ds(off + g * LANES, LANES)]
                xs = (xv - V_MIN) / (V_MAX - V_MIN) * scale
                xs = jnp.minimum(jnp.maximum(xs, 0.0), scale)
                idx = xs.astype(jnp.int32)
                w_hi = xs - idx.astype(jnp.float32)
                w_lo = 1.0 - w_hi
                b_lo = idx * EMB_DIM
                b_hi = jnp.minimum(idx + 1, NUM_EMB - 1) * EMB_DIM
                o_base = g * (LANES * EMB_DIM) + lane64

                @plsc.parallel_loop(0, EMB_DIM, unroll=8)
                def cbody(c):
                    a = plsc.load_gather(table_v, [b_lo + c])
                    b = plsc.load_gather(table_v, [b_hi + c])
                    o = w_lo * a + w_hi * b
                    plsc.store_scatter(buf_v, [o_base + c], o)

                return carry2

            lax.fori_loop(0, GROUPS, group_body, 0)
            pltpu.make_async_copy(
                buf_v,
                out_hbm.at[pl.ds((span + off) * EMB_DIM, OUT_W)],
                sem,
            ).start()

        def super_body(si, carry):
            for buf in range(2):
                ci = si * 2 + buf

                @pl.when(si > 0)
                def _wait():
                    pltpu.make_async_copy(
                        bufs[buf],
                        out_hbm.at[pl.ds(span * EMB_DIM, OUT_W)],
                        sems[buf],
                    ).wait()

                run_chunk(ci, bufs[buf], sems[buf])
            return carry

        lax.fori_loop(0, N_SUPER, super_body, 0)
        for buf in range(2):
            pltpu.make_async_copy(
                bufs[buf],
                out_hbm.at[pl.ds(span * EMB_DIM, OUT_W)],
                sems[buf],
            ).wait()

    return body


_body = _make_body()


@jax.jit
def kernel(x, embeddings):
    out = _body(x.reshape(-1), embeddings.reshape(-1))
    return out.reshape(BATCH, INPUT_DIM * EMB_DIM)


# speedup vs baseline: 6.8190x; 1.7005x over previous
import functools

import jax
import jax.numpy as jnp
from jax import lax
from jax.experimental import pallas as pl
from jax.experimental.pallas import tpu as pltpu
from jax.experimental.pallas import tpu_sc as plsc

V_MIN, V_MAX = -6.0, 6.0
BATCH, INPUT_DIM = 4096, 100
NUM_EMB, EMB_DIM = 1000, 64

NUM_CORES, NUM_SUBCORES, LANES = 2, 16, 16
NW = NUM_CORES * NUM_SUBCORES
N_ELEM = BATCH * INPUT_DIM
PER_W = N_ELEM // NW
CHUNK = 256
N_CHUNKS = PER_W // CHUNK
N_SUPER = N_CHUNKS // 2
GROUPS = CHUNK // LANES
OUT_W = CHUNK * EMB_DIM


def _make_body():
    mesh = plsc.VectorSubcoreMesh(core_axis_name="c", subcore_axis_name="s")

    @functools.partial(
        pl.kernel,
        mesh=mesh,
        out_type=jax.ShapeDtypeStruct((N_ELEM * EMB_DIM,), jnp.float32),
        scratch_types=[
            pltpu.VMEM((NUM_EMB * EMB_DIM,), jnp.float32),
            pltpu.VMEM((PER_W,), jnp.float32),
            pltpu.VMEM((OUT_W,), jnp.float32),
            pltpu.VMEM((OUT_W,), jnp.float32),
            pltpu.SemaphoreType.DMA,
            pltpu.SemaphoreType.DMA,
        ],
        compiler_params=pltpu.CompilerParams(needs_layout_passes=False),
    )
    def body(x_hbm, emb_hbm, out_hbm, table_v, x_v, out0_v, out1_v, sem0, sem1):
        wid = lax.axis_index("s") * NUM_CORES + lax.axis_index("c")
        span = wid * PER_W
        pltpu.sync_copy(emb_hbm, table_v)
        pltpu.sync_copy(x_hbm.at[pl.ds(span, PER_W)], x_v)
        lane64 = lax.iota(jnp.int32, LANES) * EMB_DIM
        scale = jnp.float32(NUM_EMB - 1)
        sems = (sem0, sem1)
        bufs = (out0_v, out1_v)

        def run_chunk(ci, buf_v, sem):
            off = ci * CHUNK

            def group_body(g, carry2):
                xv = x_v[pl.ds(off + g * LANES, LANES)]
                xs = (xv - V_MIN) / (V_MAX - V_MIN) * scale
                xs = jnp.minimum(jnp.maximum(xs, 0.0), scale)
                idx = xs.astype(jnp.int32)
                w_hi = xs - idx.astype(jnp.float32)
                w_lo = 1.0 - w_hi
                b_lo = idx
                b_hi = jnp.minimum(idx + 1, NUM_EMB - 1)
                o_base = g * (LANES * EMB_DIM) + lane64

                @plsc.parallel_loop(0, EMB_DIM, unroll=16)
                def cbody(c):
                    crow = c * NUM_EMB
                    a = plsc.load_gather(table_v, [crow + b_lo])
                    b = plsc.load_gather(table_v, [crow + b_hi])
                    o = w_lo * a + w_hi * b
                    plsc.store_scatter(buf_v, [o_base + c], o)

                return carry2

            lax.fori_loop(0, GROUPS, group_body, 0)
            pltpu.make_async_copy(
                buf_v,
                out_hbm.at[pl.ds((span + off) * EMB_DIM, OUT_W)],
                sem,
            ).start()

        def super_body(si, carry):
            for buf in range(2):
                ci = si * 2 + buf

                @pl.when(si > 0)
                def _wait():
                    pltpu.make_async_copy(
                        bufs[buf],
                        out_hbm.at[pl.ds(span * EMB_DIM, OUT_W)],
                        sems[buf],
                    ).wait()

                run_chunk(ci, bufs[buf], sems[buf])
            return carry

        lax.fori_loop(0, N_SUPER, super_body, 0)
        for buf in range(2):
            pltpu.make_async_copy(
                bufs[buf],
                out_hbm.at[pl.ds(span * EMB_DIM, OUT_W)],
                sems[buf],
            ).wait()

    return body


_body = _make_body()


@jax.jit
def kernel(x, embeddings):
    out = _body(x.reshape(-1), embeddings.T.reshape(-1))
    return out.reshape(BATCH, INPUT_DIM * EMB_DIM)


# speedup vs baseline: 14.8620x; 2.1795x over previous
import functools

import jax
import jax.numpy as jnp
from jax import lax
from jax.experimental import pallas as pl
from jax.experimental.pallas import tpu as pltpu
from jax.experimental.pallas import tpu_sc as plsc

V_MIN, V_MAX = -6.0, 6.0
BATCH, INPUT_DIM = 4096, 100
NUM_EMB, EMB_DIM = 1000, 64

NUM_CORES, NUM_SUBCORES, LANES = 2, 16, 16
NW = NUM_CORES * NUM_SUBCORES
N_ELEM = BATCH * INPUT_DIM
PER_W = N_ELEM // NW
CHUNK = 256
N_CHUNKS = PER_W // CHUNK
N_SUPER = N_CHUNKS // 2
GROUPS = CHUNK // LANES
OUT_W = CHUNK * EMB_DIM


def _make_body():
    mesh = plsc.VectorSubcoreMesh(core_axis_name="c", subcore_axis_name="s")

    @functools.partial(
        pl.kernel,
        mesh=mesh,
        out_type=jax.ShapeDtypeStruct((N_ELEM * EMB_DIM,), jnp.float32),
        scratch_types=[
            pltpu.VMEM((NUM_EMB * EMB_DIM,), jnp.float32),
            pltpu.VMEM((PER_W,), jnp.float32),
            pltpu.VMEM((OUT_W,), jnp.float32),
            pltpu.VMEM((OUT_W,), jnp.float32),
            pltpu.SemaphoreType.DMA,
            pltpu.SemaphoreType.DMA,
        ],
        compiler_params=pltpu.CompilerParams(needs_layout_passes=False),
    )
    def body(x_hbm, emb_hbm, out_hbm, table_v, x_v, out0_v, out1_v, sem0, sem1):
        wid = lax.axis_index("s") * NUM_CORES + lax.axis_index("c")
        span = wid * PER_W
        pltpu.sync_copy(emb_hbm, table_v)
        pltpu.sync_copy(x_hbm.at[pl.ds(span, PER_W)], x_v)
        lane = lax.iota(jnp.int32, LANES)
        lane64 = lane * EMB_DIM
        scale = jnp.float32(NUM_EMB - 1)
        sems = (sem0, sem1)
        bufs = (out0_v, out1_v)

        def run_chunk(ci, buf_v, sem):
            off = ci * CHUNK

            def group_body(g, carry2):
                xv = x_v[pl.ds(off + g * LANES, LANES)]
                xs = (xv - V_MIN) / (V_MAX - V_MIN) * scale
                xs = jnp.minimum(jnp.maximum(xs, 0.0), scale)
                idx = xs.astype(jnp.int32)
                w_hi = xs - idx.astype(jnp.float32)
                w_lo = 1.0 - w_hi
                b_lo = idx
                b_hi = jnp.minimum(idx + 1, NUM_EMB - 1)
                o_base = g * (LANES * EMB_DIM) + lane64

                @plsc.parallel_loop(0, EMB_DIM, unroll=16)
                def cbody(c):
                    col = (lane + c) & (EMB_DIM - 1)
                    crow = col * NUM_EMB
                    a = plsc.load_gather(table_v, [crow + b_lo])
                    b = plsc.load_gather(table_v, [crow + b_hi])
                    o = w_lo * a + w_hi * b
                    plsc.store_scatter(buf_v, [o_base + col], o)

                return carry2

            lax.fori_loop(0, GROUPS, group_body, 0)
            pltpu.make_async_copy(
                buf_v,
                out_hbm.at[pl.ds((span + off) * EMB_DIM, OUT_W)],
                sem,
            ).start()

        def super_body(si, carry):
            for buf in range(2):
                ci = si * 2 + buf

                @pl.when(si > 0)
                def _wait():
                    pltpu.make_async_copy(
                        bufs[buf],
                        out_hbm.at[pl.ds(span * EMB_DIM, OUT_W)],
                        sems[buf],
                    ).wait()

                run_chunk(ci, bufs[buf], sems[buf])
            return carry

        lax.fori_loop(0, N_SUPER, super_body, 0)
        for buf in range(2):
            pltpu.make_async_copy(
                bufs[buf],
                out_hbm.at[pl.ds(span * EMB_DIM, OUT_W)],
                sems[buf],
            ).wait()

    return body


_body = _make_body()


@jax.jit
def kernel(x, embeddings):
    out = _body(x.reshape(-1), embeddings.T.reshape(-1))
    return out.reshape(BATCH, INPUT_DIM * EMB_DIM)
